# initial kernel scaffold (unmeasured)
import jax
import jax.numpy as jnp
from jax import lax
from jax.experimental import pallas as pl
from jax.experimental.pallas import tpu as pltpu

N_DEV = 16


def kernel(Q, K, V):
    b, q, h, d = Q.shape
    _, kk, _, _ = K.shape
    scale = d ** -0.5
    Qr = Q.reshape(b, h, d)

    def body(q_ref, k_ref, v_ref, out_ref, comm_ref, send_sems, recv_sems):
        me = lax.axis_index("i")

        qv = q_ref[...] * scale
        s = jnp.sum(k_ref[...] * qv[:, None, :, :], axis=-1)
        p = jnp.exp(s)
        l_loc = jnp.sum(p, axis=1)
        o_loc = jnp.sum(p[..., None] * v_ref[...], axis=1)

        slot = comm_ref.at[me]
        slot[0:b, :, :] = o_loc
        slot[b, :, 0:h] = l_loc

        rdmas = []
        for off in range(1, N_DEV):
            tgt = lax.rem(me + off, N_DEV)
            rdma = pltpu.make_async_remote_copy(
                src_ref=comm_ref.at[me],
                dst_ref=comm_ref.at[me],
                send_sem=send_sems.at[off - 1],
                recv_sem=recv_sems.at[me],
                device_id=(tgt,),
                device_id_type=pl.DeviceIdType.MESH,
            )
            rdma.start()
            rdmas.append(rdma)

        o_acc = o_loc
        l_acc = l_loc
        for off in range(1, N_DEV):
            src = lax.rem(me + off, N_DEV)
            recv = pltpu.make_async_remote_copy(
                src_ref=comm_ref.at[me],
                dst_ref=comm_ref.at[src],
                send_sem=send_sems.at[off - 1],
                recv_sem=recv_sems.at[src],
                device_id=(me,),
                device_id_type=pl.DeviceIdType.MESH,
            )
            recv.wait_recv()
            sl = comm_ref.at[src]
            o_acc = o_acc + sl[0:b, :, :]
            l_acc = l_acc + sl[b, :, 0:h]

        out_ref[...] = o_acc / l_acc[:, :, None]

        for rdma in rdmas:
            rdma.wait_send()

    out = pl.pallas_call(
        body,
        out_shape=jax.ShapeDtypeStruct((b, h, d), jnp.float32),
        in_specs=[
            pl.BlockSpec(memory_space=pltpu.VMEM),
            pl.BlockSpec(memory_space=pltpu.VMEM),
            pl.BlockSpec(memory_space=pltpu.VMEM),
        ],
        out_specs=pl.BlockSpec(memory_space=pltpu.VMEM),
        scratch_shapes=[
            pltpu.VMEM((N_DEV, b + 1, h, d), jnp.float32),
            pltpu.SemaphoreType.DMA((N_DEV - 1,)),
            pltpu.SemaphoreType.DMA((N_DEV,)),
        ],
        compiler_params=pltpu.CompilerParams(collective_id=0),
    )(Qr, K, V)
    return out.reshape(b, 1, h, d)


# baseline (device time: 46721 ns/iter reference)
import jax
import jax.numpy as jnp
from jax import lax
from jax.experimental import pallas as pl
from jax.experimental.pallas import tpu as pltpu

N_DEV = 16


def kernel(Q, K, V):
    b, q, h, d = Q.shape
    _, kk, _, _ = K.shape
    scale = d ** -0.5
    Qr = Q.reshape(b, h, d)

    def body(q_ref, k_ref, v_ref, out_ref, comm_ref, send_sems, recv_sems):
        me = lax.axis_index("i")

        qv = q_ref[...] * scale
        s = jnp.sum(k_ref[...] * qv[:, None, :, :], axis=-1)
        p = jnp.exp(s)
        l_loc = jnp.sum(p, axis=1)
        o_loc = jnp.sum(p[..., None] * v_ref[...], axis=1)

        slot = comm_ref.at[me]
        slot[0:b, :, :] = o_loc
        slot[b, :, 0:h] = l_loc

        rdmas = []
        for off in range(1, N_DEV):
            tgt = lax.rem(me + off, N_DEV)
            rdma = pltpu.make_async_remote_copy(
                src_ref=comm_ref.at[me],
                dst_ref=comm_ref.at[me],
                send_sem=send_sems.at[off - 1],
                recv_sem=recv_sems.at[me],
                device_id=(tgt,),
                device_id_type=pl.DeviceIdType.MESH,
            )
            rdma.start()
            rdmas.append(rdma)

        o_acc = o_loc
        l_acc = l_loc
        for off in range(1, N_DEV):
            src = lax.rem(me + off, N_DEV)
            recv = pltpu.make_async_remote_copy(
                src_ref=comm_ref.at[me],
                dst_ref=comm_ref.at[src],
                send_sem=send_sems.at[off - 1],
                recv_sem=recv_sems.at[src],
                device_id=(me,),
                device_id_type=pl.DeviceIdType.MESH,
            )
            recv.wait_recv()
            sl = comm_ref.at[src]
            o_acc = o_acc + sl[0:b, :, :]
            l_acc = l_acc + sl[b, :, 0:h]

        out_ref[...] = o_acc / l_acc[:, :, None]

        for rdma in rdmas:
            rdma.wait_send()

    out = pl.pallas_call(
        body,
        out_shape=jax.ShapeDtypeStruct((b, h, d), jnp.float32),
        in_specs=[
            pl.BlockSpec(memory_space=pltpu.VMEM),
            pl.BlockSpec(memory_space=pltpu.VMEM),
            pl.BlockSpec(memory_space=pltpu.VMEM),
        ],
        out_specs=pl.BlockSpec(memory_space=pltpu.VMEM),
        scratch_shapes=[
            pltpu.VMEM((N_DEV, b + 1, h, d), jnp.float32),
            pltpu.SemaphoreType.DMA((N_DEV - 1,)),
            pltpu.SemaphoreType.DMA((N_DEV,)),
        ],
    )(Qr, K, V)
    return out.reshape(b, 1, h, d)


# device time: 33945 ns/iter; 1.3764x vs baseline; 1.3764x over previous
import jax
import jax.numpy as jnp
from jax import lax
from jax.experimental import pallas as pl
from jax.experimental.pallas import tpu as pltpu

N_DEV = 16


def kernel(Q, K, V):
    b, q, h, d = Q.shape
    kk = K.shape[1]
    scale = d ** -0.5
    Qr = Q.reshape(b, h, d)
    K2 = jnp.transpose(K, (0, 2, 3, 1))
    V2 = jnp.transpose(V, (0, 2, 3, 1))

    def body(q_ref, k_ref, v_ref, out_ref, comm_ref, send_sems, recv_sems):
        me = lax.axis_index("i")

        qv = q_ref[...] * scale
        s = jnp.sum(k_ref[...] * qv[..., None], axis=2)
        p = jnp.exp(s)
        l_loc = jnp.sum(p, axis=-1)
        o_loc = jnp.sum(p[:, :, None, :] * v_ref[...], axis=-1)

        slot = comm_ref.at[me]
        slot[0:b, :, :] = o_loc
        slot[b, :, 0:h] = l_loc

        rdmas = []
        for off in range(1, N_DEV):
            tgt = lax.rem(me + off, N_DEV)
            rdma = pltpu.make_async_remote_copy(
                src_ref=comm_ref.at[me],
                dst_ref=comm_ref.at[me],
                send_sem=send_sems.at[off - 1],
                recv_sem=recv_sems.at[me],
                device_id=(tgt,),
                device_id_type=pl.DeviceIdType.MESH,
            )
            rdma.start()
            rdmas.append(rdma)

        o_acc = o_loc
        l_acc = l_loc
        for kth in range(1, N_DEV):
            src = lax.rem(me - kth + N_DEV, N_DEV)
            recv = pltpu.make_async_remote_copy(
                src_ref=comm_ref.at[me],
                dst_ref=comm_ref.at[src],
                send_sem=send_sems.at[kth - 1],
                recv_sem=recv_sems.at[src],
                device_id=(me,),
                device_id_type=pl.DeviceIdType.MESH,
            )
            recv.wait_recv()
            sl = comm_ref.at[src]
            o_acc = o_acc + sl[0:b, :, :]
            l_acc = l_acc + sl[b, :, 0:h]

        out_ref[...] = o_acc / l_acc[:, :, None]

        for rdma in rdmas:
            rdma.wait_send()

    out = pl.pallas_call(
        body,
        out_shape=jax.ShapeDtypeStruct((b, h, d), jnp.float32),
        in_specs=[
            pl.BlockSpec(memory_space=pltpu.VMEM),
            pl.BlockSpec(memory_space=pltpu.VMEM),
            pl.BlockSpec(memory_space=pltpu.VMEM),
        ],
        out_specs=pl.BlockSpec(memory_space=pltpu.VMEM),
        scratch_shapes=[
            pltpu.VMEM((N_DEV, b + 1, h, d), jnp.float32),
            pltpu.SemaphoreType.DMA((N_DEV - 1,)),
            pltpu.SemaphoreType.DMA((N_DEV,)),
        ],
    )(Qr, K2, V2)
    return out.reshape(b, 1, h, d)


# device time: 28987 ns/iter; 1.6118x vs baseline; 1.1710x over previous
import jax
import jax.numpy as jnp
from jax import lax
from jax.experimental import pallas as pl
from jax.experimental.pallas import tpu as pltpu

N_DEV = 16


def kernel(Q, K, V):
    b, q, h, d = Q.shape
    kk = K.shape[1]
    scale = d ** -0.5
    Qr = Q.reshape(b, h, d)
    K2 = jnp.transpose(K, (0, 2, 3, 1))
    V2 = jnp.transpose(V, (0, 2, 3, 1))

    def body(q_ref, k_hbm, v_hbm, out_ref, k_ref, v_ref, comm_ref,
             send_sems, recv_sems, ready_sems, load_sems):
        me = lax.axis_index("i")

        barrier_sem = pltpu.get_barrier_semaphore()
        pl.semaphore_signal(barrier_sem, 1)
        pl.semaphore_wait(barrier_sem, 1)

        for off in range(1, N_DEV):
            tgt = lax.rem(me + off, N_DEV)
            pl.semaphore_signal(
                ready_sems.at[me], inc=1,
                device_id=(tgt,), device_id_type=pl.DeviceIdType.MESH,
            )

        k_dma = pltpu.make_async_copy(k_hbm, k_ref, load_sems.at[0])
        v_dma = pltpu.make_async_copy(v_hbm, v_ref, load_sems.at[1])
        k_dma.start()
        v_dma.start()

        qv = q_ref[...] * scale
        k_dma.wait()
        s = jnp.sum(k_ref[...] * qv[..., None], axis=2)
        p = jnp.exp(s)
        l_loc = jnp.sum(p, axis=-1)
        v_dma.wait()
        o_loc = jnp.sum(p[:, :, None, :] * v_ref[...], axis=-1)

        slot = comm_ref.at[me]
        slot[0:b, :, :] = o_loc
        slot[b, :, 0:h] = l_loc

        rdmas = []
        for off in range(1, N_DEV):
            tgt = lax.rem(me + off, N_DEV)
            pl.semaphore_wait(ready_sems.at[tgt], 1)
            rdma = pltpu.make_async_remote_copy(
                src_ref=comm_ref.at[me],
                dst_ref=comm_ref.at[me],
                send_sem=send_sems.at[off - 1],
                recv_sem=recv_sems.at[me],
                device_id=(tgt,),
                device_id_type=pl.DeviceIdType.MESH,
            )
            rdma.start()
            rdmas.append(rdma)

        o_acc = o_loc
        l_acc = l_loc
        for kth in range(1, N_DEV):
            src = lax.rem(me - kth + N_DEV, N_DEV)
            recv = pltpu.make_async_remote_copy(
                src_ref=comm_ref.at[me],
                dst_ref=comm_ref.at[src],
                send_sem=send_sems.at[kth - 1],
                recv_sem=recv_sems.at[src],
                device_id=(me,),
                device_id_type=pl.DeviceIdType.MESH,
            )
            recv.wait_recv()
            sl = comm_ref.at[src]
            o_acc = o_acc + sl[0:b, :, :]
            l_acc = l_acc + sl[b, :, 0:h]

        out_ref[...] = o_acc / l_acc[:, :, None]

        for rdma in rdmas:
            rdma.wait_send()

    out = pl.pallas_call(
        body,
        out_shape=jax.ShapeDtypeStruct((b, h, d), jnp.float32),
        in_specs=[
            pl.BlockSpec(memory_space=pltpu.VMEM),
            pl.BlockSpec(memory_space=pltpu.MemorySpace.HBM),
            pl.BlockSpec(memory_space=pltpu.MemorySpace.HBM),
        ],
        out_specs=pl.BlockSpec(memory_space=pltpu.VMEM),
        scratch_shapes=[
            pltpu.VMEM((b, h, d, kk), jnp.float32),
            pltpu.VMEM((b, h, d, kk), jnp.float32),
            pltpu.VMEM((N_DEV, b + 1, h, d), jnp.float32),
            pltpu.SemaphoreType.DMA((N_DEV - 1,)),
            pltpu.SemaphoreType.DMA((N_DEV,)),
            pltpu.SemaphoreType.REGULAR((N_DEV,)),
            pltpu.SemaphoreType.DMA((2,)),
        ],
        compiler_params=pltpu.CompilerParams(collective_id=0),
    )(Qr, K2, V2)
    return out.reshape(b, 1, h, d)


# device time: 19520 ns/iter; 2.3935x vs baseline; 1.4850x over previous
import jax
import jax.numpy as jnp
from jax import lax
from jax.experimental import pallas as pl
from jax.experimental.pallas import tpu as pltpu

N_DEV = 16


def kernel(Q, K, V):
    b, q, h, d = Q.shape
    kk = K.shape[1]
    scale = d ** -0.5
    Qr = Q.reshape(b, h, d)
    K2 = jnp.transpose(K, (0, 2, 3, 1))
    V2 = jnp.transpose(V, (0, 2, 3, 1))

    def body(q_ref, k_hbm, v_hbm, out_ref, k_ref, v_ref, comm_ref,
             send_sems, recv_sems, ready_sems, load_sems):
        me = lax.axis_index("i")

        barrier_sem = pltpu.get_barrier_semaphore()
        pl.semaphore_signal(barrier_sem, 1)
        pl.semaphore_wait(barrier_sem, 1)

        for off in range(1, N_DEV):
            tgt = lax.rem(me + off, N_DEV)
            pl.semaphore_signal(
                ready_sems.at[me], inc=1,
                device_id=(tgt,), device_id_type=pl.DeviceIdType.MESH,
            )

        k_dma = pltpu.make_async_copy(k_hbm, k_ref, load_sems.at[0])
        v_dma = pltpu.make_async_copy(v_hbm, v_ref, load_sems.at[1])
        k_dma.start()
        qv = q_ref[...] * scale
        k_dma.wait()
        v_dma.start()

        s = jnp.sum(k_ref[...] * qv[..., None], axis=2)
        p = jnp.exp(s)
        l_loc = jnp.sum(p, axis=-1)
        v_dma.wait()
        o_loc = jnp.sum(p[:, :, None, :] * v_ref[...], axis=-1)

        slot = comm_ref.at[me]
        slot[0:b, :, :] = o_loc.astype(jnp.bfloat16)
        slot[b, :, 0:h] = l_loc.astype(jnp.bfloat16)

        rdmas = []
        for off in range(1, N_DEV):
            tgt = lax.rem(me + off, N_DEV)
            pl.semaphore_wait(ready_sems.at[tgt], 1)
            rdma = pltpu.make_async_remote_copy(
                src_ref=comm_ref.at[me],
                dst_ref=comm_ref.at[me],
                send_sem=send_sems.at[off - 1],
                recv_sem=recv_sems.at[me],
                device_id=(tgt,),
                device_id_type=pl.DeviceIdType.MESH,
            )
            rdma.start()
            rdmas.append(rdma)

        acc = comm_ref.at[me][...].astype(jnp.float32)
        for kth in range(1, N_DEV):
            src = lax.rem(me - kth + N_DEV, N_DEV)
            recv = pltpu.make_async_remote_copy(
                src_ref=comm_ref.at[me],
                dst_ref=comm_ref.at[src],
                send_sem=send_sems.at[kth - 1],
                recv_sem=recv_sems.at[src],
                device_id=(me,),
                device_id_type=pl.DeviceIdType.MESH,
            )
            recv.wait_recv()
            acc = acc + comm_ref.at[src][...].astype(jnp.float32)

        out_ref[...] = acc[0:b, :, :] / acc[b, :, 0:h][:, :, None]

        for rdma in rdmas:
            rdma.wait_send()

    out = pl.pallas_call(
        body,
        out_shape=jax.ShapeDtypeStruct((b, h, d), jnp.float32),
        in_specs=[
            pl.BlockSpec(memory_space=pltpu.VMEM),
            pl.BlockSpec(memory_space=pltpu.MemorySpace.HBM),
            pl.BlockSpec(memory_space=pltpu.MemorySpace.HBM),
        ],
        out_specs=pl.BlockSpec(memory_space=pltpu.VMEM),
        scratch_shapes=[
            pltpu.VMEM((b, h, d, kk), jnp.float32),
            pltpu.VMEM((b, h, d, kk), jnp.float32),
            pltpu.VMEM((N_DEV, b + 1, h, d), jnp.bfloat16),
            pltpu.SemaphoreType.DMA((N_DEV - 1,)),
            pltpu.SemaphoreType.DMA((N_DEV,)),
            pltpu.SemaphoreType.REGULAR((N_DEV,)),
            pltpu.SemaphoreType.DMA((2,)),
        ],
        compiler_params=pltpu.CompilerParams(collective_id=0),
    )(Qr, K2, V2)
    return out.reshape(b, 1, h, d)
